# baseline (speedup 1.0000x reference)
import functools

import jax
import jax.numpy as jnp
from jax import lax
from jax.experimental import pallas as pl
from jax.experimental.pallas import tpu as pltpu
from jax.experimental.pallas import tpu_sc as plsc

N = 10000
E = 320000
D = 128
NC = 40

NSC = 2
NTEC = 16
NW = NSC * NTEC
E_PER_TILE = E // NW
CHUNK = 104
NCHUNK = 98
EPT_PAD = NCHUNK * CHUNK
NBUF = 2
NA = N + 8
ROWS_PER_TILE = 624
TAIL_ROWS = N - NTEC * ROWS_PER_TILE

_MESH = plsc.VectorSubcoreMesh(core_axis_name="c", subcore_axis_name="s")


@functools.partial(
    pl.kernel,
    out_type=jax.ShapeDtypeStruct((NSC, N, D), jnp.float32),
    mesh=_MESH,
    scratch_types=[
        pltpu.VMEM((EPT_PAD,), jnp.int32),
        pltpu.VMEM((NCHUNK, CHUNK), jnp.int32),
        pltpu.VMEM((NBUF, CHUNK, D), jnp.float32),
        pltpu.VMEM_SHARED((NA, D), jnp.float32),
        pltpu.SemaphoreType.DMA((NBUF,)),
        pltpu.SemaphoreType.DMA,
    ],
)
def _sc_aggregate(m_hbm, src_hbm, dst_hbm, zeros_hbm, out_hbm,
                  src_v, dst_v, rows_v, acc_sh, gsems, sem):
    cid = lax.axis_index("c")
    sid = lax.axis_index("s")
    wid = cid * NTEC + sid

    pltpu.async_copy(src_hbm.at[wid], src_v, sem).wait()

    def start_gather(j, b):
        pltpu.async_copy(m_hbm.at[src_v.at[pl.ds(j * CHUNK, CHUNK)]],
                         rows_v.at[b], gsems.at[b])

    def wait_gather(j, b):
        pltpu.make_async_copy(m_hbm.at[src_v.at[pl.ds(j * CHUNK, CHUNK)]],
                              rows_v.at[b], gsems.at[b]).wait()

    for b in range(NBUF):
        start_gather(b, b)

    pltpu.async_copy(dst_hbm.at[wid], dst_v, sem)

    pltpu.sync_copy(zeros_hbm, acc_sh.at[pl.ds(sid * ROWS_PER_TILE, ROWS_PER_TILE)])

    @pl.when(sid == 0)
    def _():
        pltpu.sync_copy(zeros_hbm.at[pl.ds(0, TAIL_ROWS)],
                        acc_sh.at[pl.ds(NTEC * ROWS_PER_TILE, TAIL_ROWS)])

    pltpu.make_async_copy(dst_hbm.at[wid], dst_v, sem).wait()
    plsc.subcore_barrier()

    def outer(i, _):
        g = i * NBUF
        for b in range(NBUF):
            j = g + b
            wait_gather(j, b)
            pltpu.sync_copy(rows_v.at[b], acc_sh.at[dst_v.at[j]], add=True)
            start_gather(j + NBUF, b)
        return 0

    lax.fori_loop(0, NCHUNK // NBUF - 1, outer, 0)

    for b in range(NBUF):
        j = NCHUNK - NBUF + b
        wait_gather(j, b)
        pltpu.sync_copy(rows_v.at[b], acc_sh.at[dst_v.at[j]], add=True)

    plsc.subcore_barrier()
    pltpu.sync_copy(acc_sh.at[pl.ds(sid * ROWS_PER_TILE, ROWS_PER_TILE)],
                    out_hbm.at[cid, pl.ds(sid * ROWS_PER_TILE, ROWS_PER_TILE)])

    @pl.when(sid == 0)
    def _():
        pltpu.sync_copy(acc_sh.at[pl.ds(NTEC * ROWS_PER_TILE, TAIL_ROWS)],
                        out_hbm.at[cid, pl.ds(NTEC * ROWS_PER_TILE, TAIL_ROWS)])


_ROWS_BLK = 400
_GRID = N // _ROWS_BLK


def _cmb_mm_relu_body(p0_ref, p1_ref, w_ref, b_ref, o_ref):
    s = p0_ref[...] + p1_ref[...]
    o_ref[...] = jnp.maximum(
        jnp.dot(s, w_ref[...], preferred_element_type=jnp.float32)
        + b_ref[...], 0.0)


def _cmb_out_body(p0_ref, p1_ref, w_ref, b_ref, o_ref):
    s = p0_ref[...] + p1_ref[...]
    o_ref[...] = jnp.dot(s, w_ref[...],
                         preferred_element_type=jnp.float32) + b_ref[...]


def _rows_spec(width=D):
    return pl.BlockSpec((_ROWS_BLK, width), lambda i: (i, 0))


def _full_spec(r, c):
    return pl.BlockSpec((r, c), lambda i: (0, 0))


def _tc_cmb_mm_relu(p0, p1, w, b):
    return pl.pallas_call(
        _cmb_mm_relu_body,
        grid=(_GRID,),
        in_specs=[_rows_spec(), _rows_spec(), _full_spec(D, D), _full_spec(1, D)],
        out_specs=_rows_spec(),
        out_shape=jax.ShapeDtypeStruct((N, D), jnp.float32),
    )(p0, p1, w, b)


def _tc_cmb_out(p0, p1, w, b):
    return pl.pallas_call(
        _cmb_out_body,
        grid=(_GRID,),
        in_specs=[_rows_spec(), _rows_spec(), _full_spec(D, NC), _full_spec(1, NC)],
        out_specs=_rows_spec(NC),
        out_shape=jax.ShapeDtypeStruct((N, NC), jnp.float32),
    )(p0, p1, w, b)


def kernel(features, edge_index, W1, b1, W2, b2, W3, b3):
    pad = ((0, 0), (0, EPT_PAD - E_PER_TILE))
    src = jnp.pad(edge_index[0].reshape(NW, E_PER_TILE), pad)
    dst = jnp.pad(edge_index[1].reshape(NW, E_PER_TILE), pad,
                  constant_values=N).reshape(NW, NCHUNK, CHUNK)
    zeros = jnp.zeros((ROWS_PER_TILE, D), jnp.float32)

    p1 = _sc_aggregate(features, src, dst, zeros)
    h1 = _tc_cmb_mm_relu(p1[0], p1[1], W1, b1.reshape(1, D))
    p2 = _sc_aggregate(h1, src, dst, zeros)
    h2 = _tc_cmb_mm_relu(p2[0], p2[1], W2, b2.reshape(1, D))
    p3 = _sc_aggregate(h2, src, dst, zeros)
    return _tc_cmb_out(p3[0], p3[1], W3, b3.reshape(1, NC))


# speedup vs baseline: 1.0477x; 1.0477x over previous
import functools

import jax
import jax.numpy as jnp
from jax import lax
from jax.experimental import pallas as pl
from jax.experimental.pallas import tpu as pltpu
from jax.experimental.pallas import tpu_sc as plsc

N = 10000
E = 320000
D = 128
NC = 40

NSC = 2
NTEC = 16
NW = NSC * NTEC
E_PER_TILE = E // NW
CHUNK = 96
NCHUNK = 106
EPT_PAD = NCHUNK * CHUNK
NBUF = 2
NA = N + 8
ROWS_PER_TILE = 624
TAIL_ROWS = N - NTEC * ROWS_PER_TILE

_MESH = plsc.VectorSubcoreMesh(core_axis_name="c", subcore_axis_name="s")


@functools.partial(
    pl.kernel,
    out_type=jax.ShapeDtypeStruct((NSC, N, D), jnp.float32),
    mesh=_MESH,
    scratch_types=[
        pltpu.VMEM((EPT_PAD,), jnp.int32),
        pltpu.VMEM((NCHUNK, CHUNK), jnp.int32),
        pltpu.VMEM((NBUF, CHUNK, D), jnp.float32),
        pltpu.VMEM_SHARED((NA, D), jnp.float32),
        pltpu.SemaphoreType.DMA((NBUF,)),
        pltpu.SemaphoreType.DMA,
    ],
)
def _sc_aggregate(m_hbm, src_hbm, dst_hbm, zeros_hbm, out_hbm,
                  src_v, dst_v, rows_v, acc_sh, gsems, sem):
    cid = lax.axis_index("c")
    sid = lax.axis_index("s")
    wid = cid * NTEC + sid

    pltpu.async_copy(src_hbm.at[wid], src_v, sem).wait()

    def start_gather(j, b):
        pltpu.async_copy(m_hbm.at[src_v.at[pl.ds(j * CHUNK, CHUNK)]],
                         rows_v.at[b], gsems.at[b])

    def wait_gather(j, b):
        pltpu.make_async_copy(m_hbm.at[src_v.at[pl.ds(j * CHUNK, CHUNK)]],
                              rows_v.at[b], gsems.at[b]).wait()

    for b in range(NBUF):
        start_gather(b, b)

    pltpu.async_copy(dst_hbm.at[wid], dst_v, sem)

    pltpu.sync_copy(zeros_hbm, acc_sh.at[pl.ds(sid * ROWS_PER_TILE, ROWS_PER_TILE)])

    @pl.when(sid == 0)
    def _():
        pltpu.sync_copy(zeros_hbm.at[pl.ds(0, TAIL_ROWS)],
                        acc_sh.at[pl.ds(NTEC * ROWS_PER_TILE, TAIL_ROWS)])

    pltpu.make_async_copy(dst_hbm.at[wid], dst_v, sem).wait()
    plsc.subcore_barrier()

    def outer(i, _):
        g = i * NBUF
        for b in range(NBUF):
            j = g + b
            wait_gather(j, b)
            pltpu.sync_copy(rows_v.at[b], acc_sh.at[dst_v.at[j]], add=True)
            start_gather(j + NBUF, b)
        return 0

    lax.fori_loop(0, NCHUNK // NBUF - 1, outer, 0)

    for b in range(NBUF):
        j = NCHUNK - NBUF + b
        wait_gather(j, b)
        pltpu.sync_copy(rows_v.at[b], acc_sh.at[dst_v.at[j]], add=True)

    plsc.subcore_barrier()
    pltpu.sync_copy(acc_sh.at[pl.ds(sid * ROWS_PER_TILE, ROWS_PER_TILE)],
                    out_hbm.at[cid, pl.ds(sid * ROWS_PER_TILE, ROWS_PER_TILE)])

    @pl.when(sid == 0)
    def _():
        pltpu.sync_copy(acc_sh.at[pl.ds(NTEC * ROWS_PER_TILE, TAIL_ROWS)],
                        out_hbm.at[cid, pl.ds(NTEC * ROWS_PER_TILE, TAIL_ROWS)])


_ROWS_BLK = 400
_GRID = N // _ROWS_BLK


def _cmb_mm_relu_body(p0_ref, p1_ref, w_ref, b_ref, o_ref):
    s = p0_ref[...] + p1_ref[...]
    o_ref[...] = jnp.maximum(
        jnp.dot(s, w_ref[...], preferred_element_type=jnp.float32)
        + b_ref[...], 0.0)


def _cmb_out_body(p0_ref, p1_ref, w_ref, b_ref, o_ref):
    s = p0_ref[...] + p1_ref[...]
    o_ref[...] = jnp.dot(s, w_ref[...],
                         preferred_element_type=jnp.float32) + b_ref[...]


def _rows_spec(width=D):
    return pl.BlockSpec((_ROWS_BLK, width), lambda i: (i, 0))


def _full_spec(r, c):
    return pl.BlockSpec((r, c), lambda i: (0, 0))


def _tc_cmb_mm_relu(p0, p1, w, b):
    return pl.pallas_call(
        _cmb_mm_relu_body,
        grid=(_GRID,),
        in_specs=[_rows_spec(), _rows_spec(), _full_spec(D, D), _full_spec(1, D)],
        out_specs=_rows_spec(),
        out_shape=jax.ShapeDtypeStruct((N, D), jnp.float32),
    )(p0, p1, w, b)


def _tc_cmb_out(p0, p1, w, b):
    return pl.pallas_call(
        _cmb_out_body,
        grid=(_GRID,),
        in_specs=[_rows_spec(), _rows_spec(), _full_spec(D, NC), _full_spec(1, NC)],
        out_specs=_rows_spec(NC),
        out_shape=jax.ShapeDtypeStruct((N, NC), jnp.float32),
    )(p0, p1, w, b)


def kernel(features, edge_index, W1, b1, W2, b2, W3, b3):
    pad = ((0, 0), (0, EPT_PAD - E_PER_TILE))
    src = jnp.pad(edge_index[0].reshape(NW, E_PER_TILE), pad)
    dst = jnp.pad(edge_index[1].reshape(NW, E_PER_TILE), pad,
                  constant_values=N).reshape(NW, NCHUNK, CHUNK)
    zeros = jnp.zeros((ROWS_PER_TILE, D), jnp.float32)

    p1 = _sc_aggregate(features, src, dst, zeros)
    h1 = _tc_cmb_mm_relu(p1[0], p1[1], W1, b1.reshape(1, D))
    p2 = _sc_aggregate(h1, src, dst, zeros)
    h2 = _tc_cmb_mm_relu(p2[0], p2[1], W2, b2.reshape(1, D))
    p3 = _sc_aggregate(h2, src, dst, zeros)
    return _tc_cmb_out(p3[0], p3[1], W3, b3.reshape(1, NC))


# speedup vs baseline: 2.4170x; 2.3070x over previous
import functools

import jax
import jax.numpy as jnp
from jax import lax
from jax.experimental import pallas as pl
from jax.experimental.pallas import tpu as pltpu
from jax.experimental.pallas import tpu_sc as plsc

N = 10000
E = 320000
D = 128
NC = 40

NSC = 2
NTEC = 16
NW = NSC * NTEC
E_PER_TILE = E // NW
CHUNK = 80
NCHUNK = E_PER_TILE // CHUNK
NBUF = 2
ROWS_PER_TILE = 624
TAIL_ROWS = N - NTEC * ROWS_PER_TILE

_MESH = plsc.VectorSubcoreMesh(core_axis_name="c", subcore_axis_name="s")


@functools.partial(
    pl.kernel,
    out_type=jax.ShapeDtypeStruct((NSC, N, D), jnp.float32),
    mesh=_MESH,
    scratch_types=[
        pltpu.VMEM((E_PER_TILE,), jnp.int32),
        pltpu.VMEM((NCHUNK, CHUNK), jnp.int32),
        pltpu.VMEM((NBUF, CHUNK, D), jnp.float32),
        pltpu.VMEM_SHARED((N, D), jnp.float32),
        pltpu.SemaphoreType.DMA((NBUF,)),
        pltpu.SemaphoreType.DMA,
    ],
)
def _sc_aggregate(m_hbm, src_hbm, dst_hbm, zeros_hbm, out_hbm,
                  src_v, dst_v, rows_v, acc_sh, gsems, sem):
    cid = lax.axis_index("c")
    sid = lax.axis_index("s")
    wid = cid * NTEC + sid

    pltpu.async_copy(src_hbm.at[wid], src_v, sem).wait()

    def start_gather(j, b):
        pltpu.async_copy(m_hbm.at[src_v.at[pl.ds(j * CHUNK, CHUNK)]],
                         rows_v.at[b], gsems.at[b])

    def wait_gather(j, b):
        pltpu.make_async_copy(m_hbm.at[src_v.at[pl.ds(j * CHUNK, CHUNK)]],
                              rows_v.at[b], gsems.at[b]).wait()

    for b in range(NBUF):
        start_gather(b, b)

    pltpu.async_copy(dst_hbm.at[wid], dst_v, sem)

    pltpu.sync_copy(zeros_hbm, acc_sh.at[pl.ds(sid * ROWS_PER_TILE, ROWS_PER_TILE)])

    @pl.when(sid == 0)
    def _():
        pltpu.sync_copy(zeros_hbm.at[pl.ds(0, TAIL_ROWS)],
                        acc_sh.at[pl.ds(NTEC * ROWS_PER_TILE, TAIL_ROWS)])

    pltpu.make_async_copy(dst_hbm.at[wid], dst_v, sem).wait()
    plsc.subcore_barrier()

    def outer(i, _):
        g = i * NBUF
        for b in range(NBUF):
            j = g + b
            wait_gather(j, b)
            pltpu.sync_copy(rows_v.at[b], acc_sh.at[dst_v.at[j]], add=True)
            start_gather(j + NBUF, b)
        return 0

    lax.fori_loop(0, (NCHUNK - 1) // NBUF - 1, outer, 0)

    for b in range(NBUF):
        j = NCHUNK - 1 - NBUF + b
        wait_gather(j, b)
        pltpu.sync_copy(rows_v.at[b], acc_sh.at[dst_v.at[j]], add=True)

    start_gather(NCHUNK - 1, 0)
    wait_gather(NCHUNK - 1, 0)
    pltpu.sync_copy(rows_v.at[0], acc_sh.at[dst_v.at[NCHUNK - 1]], add=True)

    plsc.subcore_barrier()
    pltpu.sync_copy(acc_sh.at[pl.ds(sid * ROWS_PER_TILE, ROWS_PER_TILE)],
                    out_hbm.at[cid, pl.ds(sid * ROWS_PER_TILE, ROWS_PER_TILE)])

    @pl.when(sid == 0)
    def _():
        pltpu.sync_copy(acc_sh.at[pl.ds(NTEC * ROWS_PER_TILE, TAIL_ROWS)],
                        out_hbm.at[cid, pl.ds(NTEC * ROWS_PER_TILE, TAIL_ROWS)])


_ROWS_BLK = 400
_GRID = N // _ROWS_BLK


def _cmb_mm_relu_body(p0_ref, p1_ref, w_ref, b_ref, o_ref):
    s = p0_ref[...] + p1_ref[...]
    o_ref[...] = jnp.maximum(
        jnp.dot(s, w_ref[...], preferred_element_type=jnp.float32)
        + b_ref[...], 0.0)


def _cmb_out_body(p0_ref, p1_ref, w_ref, b_ref, o_ref):
    s = p0_ref[...] + p1_ref[...]
    o_ref[...] = jnp.dot(s, w_ref[...],
                         preferred_element_type=jnp.float32) + b_ref[...]


def _rows_spec(width=D):
    return pl.BlockSpec((_ROWS_BLK, width), lambda i: (i, 0))


def _full_spec(r, c):
    return pl.BlockSpec((r, c), lambda i: (0, 0))


def _tc_cmb_mm_relu(p0, p1, w, b):
    return pl.pallas_call(
        _cmb_mm_relu_body,
        grid=(_GRID,),
        in_specs=[_rows_spec(), _rows_spec(), _full_spec(D, D), _full_spec(1, D)],
        out_specs=_rows_spec(),
        out_shape=jax.ShapeDtypeStruct((N, D), jnp.float32),
    )(p0, p1, w, b)


def _tc_cmb_out(p0, p1, w, b):
    return pl.pallas_call(
        _cmb_out_body,
        grid=(_GRID,),
        in_specs=[_rows_spec(), _rows_spec(), _full_spec(D, NC), _full_spec(1, NC)],
        out_specs=_rows_spec(NC),
        out_shape=jax.ShapeDtypeStruct((N, NC), jnp.float32),
    )(p0, p1, w, b)


def kernel(features, edge_index, W1, b1, W2, b2, W3, b3):
    src = edge_index[0].reshape(NW, E_PER_TILE)
    dst = edge_index[1].reshape(NW, NCHUNK, CHUNK)
    zeros = jnp.zeros((ROWS_PER_TILE, D), jnp.float32)

    p1 = _sc_aggregate(features, src, dst, zeros)
    h1 = _tc_cmb_mm_relu(p1[0], p1[1], W1, b1.reshape(1, D))
    p2 = _sc_aggregate(h1, src, dst, zeros)
    h2 = _tc_cmb_mm_relu(p2[0], p2[1], W2, b2.reshape(1, D))
    p3 = _sc_aggregate(h2, src, dst, zeros)
    return _tc_cmb_out(p3[0], p3[1], W3, b3.reshape(1, NC))
